# baseline (speedup 1.0000x reference)
import functools

import jax
import jax.numpy as jnp
from jax import lax
from jax.experimental import pallas as pl
from jax.experimental.pallas import tpu as pltpu
from jax.experimental.pallas import tpu_sc as plsc

VOCAB = 1_000_000
D = 64
BATCH = 4096
SEQ = 200
B = BATCH * SEQ
SCALE = 8.0

NC, NS, L = 2, 16, 16
NW = NC * NS
ROWS_PER_W = B // NW
SUB = 128
CHUNK = 1024
NSUB = CHUNK // SUB
NCHUNKS = ROWS_PER_W // CHUNK


@functools.partial(
    pl.kernel,
    out_type=jax.ShapeDtypeStruct((B, D), jnp.float32),
    mesh=plsc.VectorSubcoreMesh(
        core_axis_name="c", subcore_axis_name="s",
        num_cores=NC, num_subcores=NS),
    scratch_types=[
        pltpu.VMEM((NSUB, SUB), jnp.int32),
        pltpu.VMEM((CHUNK, D), jnp.float32),
        pltpu.SemaphoreType.DMA,
    ],
    compiler_params=pltpu.CompilerParams(use_tc_tiling_on_sc=False),
)
def _embed_sc(tok_hbm, table_hbm, out_hbm, idx_v, rows_v, sem):
    wid = lax.axis_index("s") * NC + lax.axis_index("c")
    row0 = wid * ROWS_PER_W

    @pl.loop(0, NCHUNKS)
    def _chunk(c):
        base = pl.multiple_of(row0 + c * CHUNK, CHUNK)
        pltpu.sync_copy(
            tok_hbm.at[pl.ds(pl.multiple_of(base // SUB, NSUB), NSUB)], idx_v)
        for j in range(NSUB):
            pltpu.async_copy(
                table_hbm.at[idx_v.at[j]],
                rows_v.at[pl.ds(j * SUB, SUB)],
                sem,
            ).wait()

        @pl.loop(0, CHUNK)
        def _scale(i):
            for k in range(D // L):
                sl = pl.ds(k * L, L)
                rows_v[i, sl] = rows_v[i, sl] * SCALE

        pltpu.sync_copy(rows_v, out_hbm.at[pl.ds(base, CHUNK)])


def kernel(tokens, table):
    tok = tokens.astype(jnp.int32).reshape(B // SUB, SUB)
    out = _embed_sc(tok, table)
    return out.reshape(tokens.shape + (D,))
